# baseline (speedup 1.0000x reference)
import functools

import jax
import jax.numpy as jnp
from jax.experimental import pallas as pl
from jax.experimental.pallas import tpu as pltpu

_TAPS = {(0, 0): (0,), (0, 1): (1, 2), (1, 0): (0, 1), (1, 1): (2,)}


def _phase_rhs(w_oihw):
    co, ci = w_oihw.shape[0], w_oihw.shape[1]
    import numpy as np
    fold = np.zeros((2, 2, 3, 2, 3, 3), np.float32)
    for e in range(2):
        for a in range(2):
            for c in range(3):
                for f in range(2):
                    b = c - f
                    if b not in (0, 1):
                        continue
                    for kh in _TAPS[(e, a)]:
                        for kw in _TAPS[(f, b)]:
                            fold[e, a, c, f, kh, kw] = 1.0
    rhs = jnp.einsum('eacfhw,oihw->eacifo', jnp.asarray(fold),
                     w_oihw.astype(jnp.float32))
    return rhs.reshape(2, 6 * ci, 2 * co).astype(jnp.bfloat16)


def _upconv_step(xm_ref, xo_ref, w_ref, b_ref, o_ref):
    th = o_ref.shape[1]
    W = o_ref.shape[3] // 2
    c = o_ref.shape[4]
    t = pl.program_id(1)

    xb = xm_ref[0].astype(jnp.bfloat16)
    prev = jnp.where(t == 1, xo_ref[0, th - 1:th].astype(jnp.bfloat16), 0)
    nxt = jnp.where(t == 0, xo_ref[0, 0:1].astype(jnp.bfloat16), 0)
    xp = jnp.concatenate([prev, xb, nxt], axis=0)
    zcol = jnp.zeros((th + 2, 1, c), jnp.bfloat16)
    sh = [
        jnp.concatenate([zcol, xp[:, :W - 1, :]], axis=1),
        xp,
        jnp.concatenate([xp[:, 1:, :], zcol], axis=1),
    ]
    bias = b_ref[...]

    for e in range(2):
        lhs = jnp.concatenate(
            [sh[cc][e + a:e + a + th].reshape(th * W, -1)
             for a in range(2) for cc in range(3)], axis=1)
        acc = jnp.dot(lhs, w_ref[e], preferred_element_type=jnp.float32)
        acc = acc + bias
        for f in range(2):
            o_ref[0, :, e, pl.ds(f, W, 2), :] = (
                acc[:, f * c:(f + 1) * c].reshape(th, W, c))


def _fused_upconv_nhwc(x_nhwc, w_oihw, bias):
    n, h, w, c = x_nhwc.shape
    w_m = _phase_rhs(w_oihw)
    b2 = jnp.tile(bias.reshape(1, c), (1, 2))

    th = h // 2
    out5 = pl.pallas_call(
        _upconv_step,
        out_shape=jax.ShapeDtypeStruct((n, h, 2, 2 * w, c), x_nhwc.dtype),
        grid=(n, 2),
        in_specs=[
            pl.BlockSpec((1, th, w, c), lambda i, t: (i, t, 0, 0)),
            pl.BlockSpec((1, th, w, c), lambda i, t: (i, 1 - t, 0, 0)),
            pl.BlockSpec((2, 6 * c, 2 * c), lambda i, t: (0, 0, 0)),
            pl.BlockSpec((1, 2 * c), lambda i, t: (0, 0)),
        ],
        out_specs=pl.BlockSpec((1, th, 2, 2 * w, c),
                               lambda i, t: (i, t, 0, 0, 0)),
        compiler_params=pltpu.CompilerParams(
            dimension_semantics=("parallel", "parallel"),
            vmem_limit_bytes=64 << 20,
        ),
    )(x_nhwc, x_nhwc, w_m, b2)

    return out5.reshape(n, 2 * h, 2 * w, c)


@jax.jit
def kernel(x_nchw, w_oihw, bias):
    out_nhwc = _fused_upconv_nhwc(
        jnp.transpose(x_nchw, (0, 2, 3, 1)), w_oihw, bias)
    return jnp.transpose(out_nhwc, (0, 3, 1, 2))


# speedup vs baseline: 1.0933x; 1.0933x over previous
import functools

import jax
import jax.numpy as jnp
from jax.experimental import pallas as pl
from jax.experimental.pallas import tpu as pltpu

_TAPS = {(0, 0): (0,), (0, 1): (1, 2), (1, 0): (0, 1), (1, 1): (2,)}


def _phase_rhs(w_oihw):
    co, ci = w_oihw.shape[0], w_oihw.shape[1]
    import numpy as np
    fold = np.zeros((2, 2, 3, 2, 3, 3), np.float32)
    for e in range(2):
        for a in range(2):
            for c in range(3):
                for f in range(2):
                    b = c - f
                    if b not in (0, 1):
                        continue
                    for kh in _TAPS[(e, a)]:
                        for kw in _TAPS[(f, b)]:
                            fold[e, a, c, f, kh, kw] = 1.0
    rhs = jnp.einsum('eacfhw,oihw->eacifo', jnp.asarray(fold),
                     w_oihw.astype(jnp.float32))
    return rhs.reshape(2, 6 * ci, 2 * co).astype(jnp.bfloat16)


def _upconv_step(xm_ref, w_ref, b_ref, o_ref):
    th = o_ref.shape[1]
    W = o_ref.shape[3] // 2
    c = o_ref.shape[4]

    xb = xm_ref[0].astype(jnp.bfloat16)
    zrow = jnp.zeros((1, W, c), jnp.bfloat16)
    xp = jnp.concatenate([zrow, xb, zrow], axis=0)
    zcol = jnp.zeros((th + 2, 1, c), jnp.bfloat16)
    sh = [
        jnp.concatenate([zcol, xp[:, :W - 1, :]], axis=1),
        xp,
        jnp.concatenate([xp[:, 1:, :], zcol], axis=1),
    ]
    bias = b_ref[...]

    for e in range(2):
        lhs = jnp.concatenate(
            [sh[cc][e + a:e + a + th].reshape(th * W, -1)
             for a in range(2) for cc in range(3)], axis=1)
        acc = jnp.dot(lhs, w_ref[e], preferred_element_type=jnp.float32)
        acc = acc + bias
        for f in range(2):
            o_ref[0, :, e, pl.ds(f, W, 2), :] = (
                acc[:, f * c:(f + 1) * c].reshape(th, W, c))


def _fused_upconv_nhwc(x_nhwc, w_oihw, bias):
    n, h, w, c = x_nhwc.shape
    w_m = _phase_rhs(w_oihw)
    b2 = jnp.tile(bias.reshape(1, c), (1, 2))

    out5 = pl.pallas_call(
        _upconv_step,
        out_shape=jax.ShapeDtypeStruct((n, h, 2, 2 * w, c), x_nhwc.dtype),
        grid=(n,),
        in_specs=[
            pl.BlockSpec((1, h, w, c), lambda i: (i, 0, 0, 0)),
            pl.BlockSpec((2, 6 * c, 2 * c), lambda i: (0, 0, 0)),
            pl.BlockSpec((1, 2 * c), lambda i: (0, 0)),
        ],
        out_specs=pl.BlockSpec((1, h, 2, 2 * w, c),
                               lambda i: (i, 0, 0, 0, 0)),
        compiler_params=pltpu.CompilerParams(
            dimension_semantics=("parallel",),
            vmem_limit_bytes=64 << 20,
        ),
    )(x_nhwc, w_m, b2)

    return out5.reshape(n, 2 * h, 2 * w, c)


@jax.jit
def kernel(x_nchw, w_oihw, bias):
    out_nhwc = _fused_upconv_nhwc(
        jnp.transpose(x_nchw, (0, 2, 3, 1)), w_oihw, bias)
    return jnp.transpose(out_nhwc, (0, 3, 1, 2))
